# baseline (speedup 1.0000x reference)
import functools

import jax
import jax.numpy as jnp
from jax import lax
from jax.experimental import pallas as pl
from jax.experimental.pallas import tpu as pltpu
from jax.experimental.pallas import tpu_sc as plsc

B, T = 1024, 50
NUM_FIELDS = 26
CONT_SIZE = 13
INPUT_SIZE = NUM_FIELDS + CONT_SIZE
VOCAB = 100000
D_MODEL = 32

N = B * T
NC, NS, LANES = 2, 16, 16
NW = NC * NS
TPW = N // NW
CT = 64
NCHUNK = TPW // CT
RPC = CT * NUM_FIELDS
GSUB = 128
NSUB = RPC // GSUB


def _tree_sum(vals):
    while len(vals) > 1:
        nxt = [vals[k] + vals[k + 1] for k in range(0, len(vals) - 1, 2)]
        if len(vals) % 2:
            nxt.append(vals[-1])
        vals = nxt
    return vals[0]


def _sc_embed_body(x_hbm, tab_hbm, emb_hbm, xv, rid, rows, emb_v, sem):
    wid = lax.axis_index("s") * NC + lax.axis_index("c")
    base0 = wid * TPW
    iota = lax.iota(jnp.int32, LANES)

    @pl.loop(0, NCHUNK)
    def _chunk(g):
        base = base0 + g * CT
        pltpu.sync_copy(x_hbm.at[pl.ds(base * INPUT_SIZE, CT * INPUT_SIZE)], xv)

        for i in range(NUM_FIELDS):
            off = jnp.full((LANES,), i * (VOCAB + 1), jnp.int32)
            for c16 in range(CT // LANES):
                src = (iota + c16 * LANES) * INPUT_SIZE + i
                r = plsc.load_gather(xv, [src])
                f = i * CT + c16 * LANES
                rid[f // GSUB, pl.ds(f % GSUB, LANES)] = r + off

        descs = [
            pltpu.async_copy(
                tab_hbm.at[rid.at[j]],
                rows.at[pl.ds(j * GSUB, GSUB)],
                sem,
            )
            for j in range(NSUB)
        ]
        for d in descs:
            d.wait()

        @pl.loop(0, CT)
        def _acc(c):
            for half in range(2):
                vals = [
                    rows[i * CT + c, pl.ds(half * LANES, LANES)]
                    for i in range(NUM_FIELDS)
                ]
                emb_v[c, pl.ds(half * LANES, LANES)] = _tree_sum(vals)

        pltpu.sync_copy(emb_v, emb_hbm.at[pl.ds(base, CT)])


_sc_embed = functools.partial(
    pl.kernel,
    out_type=jax.ShapeDtypeStruct((N, D_MODEL), jnp.float32),
    mesh=plsc.VectorSubcoreMesh(
        core_axis_name="c", subcore_axis_name="s", num_cores=NC, num_subcores=NS
    ),
    compiler_params=pltpu.CompilerParams(
        needs_layout_passes=False, use_tc_tiling_on_sc=False
    ),
    scratch_types=[
        pltpu.VMEM((CT * INPUT_SIZE,), jnp.int32),
        pltpu.VMEM((NSUB, GSUB), jnp.int32),
        pltpu.VMEM((RPC, D_MODEL), jnp.float32),
        pltpu.VMEM((CT, D_MODEL), jnp.float32),
        pltpu.SemaphoreType.DMA,
    ],
)(_sc_embed_body)


ROWS_BLK = 2048


def _tc_proj_body(emb_ref, x_ref, w_ref, b_ref, out_ref):
    emb = emb_ref[...]
    cont = x_ref[:, NUM_FIELDS:].astype(jnp.float32)
    w = w_ref[...]
    out = lax.dot_general(
        emb, w[:, :D_MODEL], (((1,), (1,)), ((), ())),
        preferred_element_type=jnp.float32,
    )
    out += lax.dot_general(
        cont, w[:, D_MODEL:], (((1,), (1,)), ((), ())),
        preferred_element_type=jnp.float32,
    )
    out_ref[...] = out + b_ref[...]


def _tc_proj(emb, x2d, W, b2d):
    return pl.pallas_call(
        _tc_proj_body,
        grid=(N // ROWS_BLK,),
        in_specs=[
            pl.BlockSpec((ROWS_BLK, D_MODEL), lambda i: (i, 0)),
            pl.BlockSpec((ROWS_BLK, INPUT_SIZE), lambda i: (i, 0)),
            pl.BlockSpec((D_MODEL, D_MODEL + CONT_SIZE), lambda i: (0, 0)),
            pl.BlockSpec((1, D_MODEL), lambda i: (0, 0)),
        ],
        out_specs=pl.BlockSpec((ROWS_BLK, D_MODEL), lambda i: (i, 0)),
        out_shape=jax.ShapeDtypeStruct((N, D_MODEL), jnp.float32),
    )(emb, x2d, W, b2d)


def kernel(x, tables, W, b):
    x2d = x.reshape(N, INPUT_SIZE).astype(jnp.int32)
    tab = tables.reshape(NUM_FIELDS * (VOCAB + 1), D_MODEL)
    emb = _sc_embed(x2d.reshape(N * INPUT_SIZE), tab)
    out2d = _tc_proj(emb, x2d, W, b.reshape(1, D_MODEL))
    return out2d.reshape(B, T, D_MODEL)


# speedup vs baseline: 7.3592x; 7.3592x over previous
import functools

import jax
import jax.numpy as jnp
from jax import lax
from jax.experimental import pallas as pl
from jax.experimental.pallas import tpu as pltpu
from jax.experimental.pallas import tpu_sc as plsc

B, T = 1024, 50
NUM_FIELDS = 26
CONT_SIZE = 13
INPUT_SIZE = NUM_FIELDS + CONT_SIZE
VOCAB = 100000
D_MODEL = 32

N = B * T
NC, NS, LANES = 2, 16, 16
NW = NC * NS
TPW = N // NW

VCHUNK = 2048
NVC = 49
VP = NVC * VCHUNK
PROWS = NUM_FIELDS * VP
PD = 128

CT = 16
NCHUNK = TPW // CT
RPC = CT * NUM_FIELDS
GSUB = 32
NSUB = RPC // GSUB


def _tree_sum(vals):
    while len(vals) > 1:
        nxt = [vals[k] + vals[k + 1] for k in range(0, len(vals) - 1, 2)]
        if len(vals) % 2:
            nxt.append(vals[-1])
        vals = nxt
    return vals[0]


def _proj_body(tabT_ref, w_ref, out_ref):
    blk = tabT_ref[0]
    out_ref[...] = lax.dot_general(
        blk, w_ref[...], (((0,), (1,)), ((), ())),
        preferred_element_type=jnp.float32,
    )


def _tc_project(tabT, w128):
    return pl.pallas_call(
        _proj_body,
        grid=(NUM_FIELDS, NVC),
        in_specs=[
            pl.BlockSpec((1, D_MODEL, VCHUNK), lambda i, c: (i, 0, c)),
            pl.BlockSpec((PD, D_MODEL), lambda i, c: (0, 0)),
        ],
        out_specs=pl.BlockSpec((VCHUNK, PD), lambda i, c: (i * NVC + c, 0)),
        out_shape=jax.ShapeDtypeStruct((PROWS, PD), jnp.float32),
    )(tabT, w128)


def _sc_embed_body(x_hbm, tab_hbm, emb_hbm, xv, rid, rows, emb_v, sem):
    wid = lax.axis_index("s") * NC + lax.axis_index("c")
    base0 = wid * TPW
    iota = lax.iota(jnp.int32, LANES)

    @pl.loop(0, NCHUNK)
    def _chunk(g):
        base = base0 + g * CT
        pltpu.sync_copy(x_hbm.at[pl.ds(base * INPUT_SIZE, CT * INPUT_SIZE)], xv)

        for i in range(NUM_FIELDS):
            off = jnp.full((LANES,), i * VP, jnp.int32)
            src = iota * INPUT_SIZE + i
            r = plsc.load_gather(xv, [src])
            f = i * CT
            rid[f // GSUB, pl.ds(f % GSUB, LANES)] = r + off

        descs = [
            pltpu.async_copy(
                tab_hbm.at[rid.at[j]],
                rows.at[pl.ds(j * GSUB, GSUB)],
                sem,
            )
            for j in range(NSUB)
        ]
        for d in descs:
            d.wait()

        @pl.loop(0, CT)
        def _acc(c):
            for half in range(2):
                vals = [
                    rows[i * CT + c, pl.ds(half * LANES, LANES)]
                    for i in range(NUM_FIELDS)
                ]
                emb_v[c, pl.ds(half * LANES, LANES)] = _tree_sum(vals)

        pltpu.sync_copy(emb_v, emb_hbm.at[pl.ds(base, CT)])


_sc_embed = functools.partial(
    pl.kernel,
    out_type=jax.ShapeDtypeStruct((N, D_MODEL), jnp.float32),
    mesh=plsc.VectorSubcoreMesh(
        core_axis_name="c", subcore_axis_name="s", num_cores=NC, num_subcores=NS
    ),
    compiler_params=pltpu.CompilerParams(
        needs_layout_passes=False, use_tc_tiling_on_sc=True
    ),
    scratch_types=[
        pltpu.VMEM((CT * INPUT_SIZE,), jnp.int32),
        pltpu.VMEM((NSUB, GSUB), jnp.int32),
        pltpu.VMEM((RPC, PD), jnp.float32),
        pltpu.VMEM((CT, D_MODEL), jnp.float32),
        pltpu.SemaphoreType.DMA,
    ],
)(_sc_embed_body)


ROWS_BLK = 2048


def _tc_cont_body(s_ref, x_ref, w_ref, b_ref, out_ref):
    cont = x_ref[:, NUM_FIELDS:].astype(jnp.float32)
    out = lax.dot_general(
        cont, w_ref[:, D_MODEL:], (((1,), (1,)), ((), ())),
        preferred_element_type=jnp.float32,
    )
    out_ref[...] = out + s_ref[...] + b_ref[...]


def _tc_cont(s, x2d, W, b2d):
    return pl.pallas_call(
        _tc_cont_body,
        grid=(N // ROWS_BLK,),
        in_specs=[
            pl.BlockSpec((ROWS_BLK, D_MODEL), lambda i: (i, 0)),
            pl.BlockSpec((ROWS_BLK, INPUT_SIZE), lambda i: (i, 0)),
            pl.BlockSpec((D_MODEL, D_MODEL + CONT_SIZE), lambda i: (0, 0)),
            pl.BlockSpec((1, D_MODEL), lambda i: (0, 0)),
        ],
        out_specs=pl.BlockSpec((ROWS_BLK, D_MODEL), lambda i: (i, 0)),
        out_shape=jax.ShapeDtypeStruct((N, D_MODEL), jnp.float32),
    )(s, x2d, W, b2d)


def kernel(x, tables, W, b):
    x2d = x.reshape(N, INPUT_SIZE).astype(jnp.int32)
    tabT = tables.transpose(0, 2, 1)
    w128 = jnp.pad(W[:, :D_MODEL], ((0, PD - D_MODEL), (0, 0)))
    proj = _tc_project(tabT, w128)
    s = _sc_embed(x2d.reshape(N * INPUT_SIZE), proj)
    out2d = _tc_cont(s, x2d, W, b.reshape(1, D_MODEL))
    return out2d.reshape(B, T, D_MODEL)
